# Initial kernel scaffold; baseline (speedup 1.0000x reference)
import jax
import jax.numpy as jnp
from jax.experimental import pallas as pl


def kernel(x, edge_index, W1, b1, W2, b2):
    raise NotImplementedError("write your pallas kernel here")


# speedup vs baseline: 15.8296x; 15.8296x over previous
import functools

import jax
import jax.numpy as jnp
from jax import lax
from jax.experimental import pallas as pl
from jax.experimental.pallas import tpu as pltpu
from jax.experimental.pallas import tpu_sc as plsc

N = 10000
NPAD = 10240
NW = 32
K = 128
CH = 79
EPER = CH * K
EPAD = NW * EPER
TROWS = NPAD // 16
DEGW = 16

_MESH = plsc.VectorSubcoreMesh(
    core_axis_name="c", subcore_axis_name="s", num_cores=2, num_subcores=16
)

_SC_PARAMS = pltpu.CompilerParams(use_tc_tiling_on_sc=False)


@functools.partial(
    pl.kernel,
    out_type=jax.ShapeDtypeStruct((2, NPAD, DEGW), jnp.float32),
    mesh=_MESH,
    scratch_types=[
        pltpu.VMEM((CH, K), jnp.int32),
        pltpu.VMEM((K, DEGW), jnp.float32),
        pltpu.VMEM_SHARED((NPAD, DEGW), jnp.float32),
    ],
    compiler_params=_SC_PARAMS,
)
def _deg_kernel(ec_hbm, ones_hbm, zz_hbm, out_hbm, col_v, obuf, acc):
    cc = lax.axis_index("c")
    ss = lax.axis_index("s")
    wid = cc * 16 + ss
    pltpu.sync_copy(zz_hbm.at[pl.ds(ss * TROWS, TROWS)],
                    acc.at[pl.ds(ss * TROWS, TROWS)])
    pltpu.sync_copy(ones_hbm, obuf)
    pltpu.sync_copy(ec_hbm.at[wid], col_v)
    plsc.subcore_barrier()

    def step(j, carry):
        pltpu.sync_copy(obuf, acc.at[col_v.at[j]], add=True)
        return carry

    lax.fori_loop(0, CH, step, 0)
    plsc.subcore_barrier()
    pltpu.sync_copy(acc.at[pl.ds(ss * TROWS, TROWS)],
                    out_hbm.at[cc, pl.ds(ss * TROWS, TROWS)])


def _make_agg(F):
    @functools.partial(
        pl.kernel,
        out_type=jax.ShapeDtypeStruct((2, NPAD, F), jnp.float32),
        mesh=_MESH,
        scratch_types=[
            pltpu.VMEM((CH, K), jnp.int32),
            pltpu.VMEM((CH, K), jnp.int32),
            pltpu.VMEM((K, F), jnp.float32),
            pltpu.VMEM_SHARED((NPAD, F), jnp.float32),
            pltpu.SemaphoreType.DMA,
        ],
        compiler_params=_SC_PARAMS,
    )
    def agg(h_hbm, er_hbm, ec_hbm, zz_hbm, out_hbm, row_v, col_v, gbuf, acc,
            sem):
        cc = lax.axis_index("c")
        ss = lax.axis_index("s")
        wid = cc * 16 + ss
        pltpu.sync_copy(zz_hbm.at[pl.ds(ss * TROWS, TROWS)],
                        acc.at[pl.ds(ss * TROWS, TROWS)])
        pltpu.sync_copy(er_hbm.at[wid], row_v)
        pltpu.sync_copy(ec_hbm.at[wid], col_v)
        plsc.subcore_barrier()

        def step(j, carry):
            pltpu.async_copy(h_hbm.at[row_v.at[j]], gbuf, sem).wait()
            pltpu.sync_copy(gbuf, acc.at[col_v.at[j]], add=True)
            return carry

        lax.fori_loop(0, CH, step, 0)
        plsc.subcore_barrier()
        pltpu.sync_copy(acc.at[pl.ds(ss * TROWS, TROWS)],
                        out_hbm.at[cc, pl.ds(ss * TROWS, TROWS)])

    return agg


_agg128 = _make_agg(128)
_agg64 = _make_agg(64)


def _tc1_body(degp, x, w1, h1p_o, dinv_o):
    deg = degp[0, :, 0:1] + degp[1, :, 0:1] + 1.0
    dinv = lax.rsqrt(deg)
    h = jnp.dot(x[...], w1[...], preferred_element_type=jnp.float32)
    h1p_o[...] = h * dinv
    dinv_o[...] = dinv


def _tc2_body(p, h1p, dinv, b1, w2, h2p_o):
    s = p[0] + p[1] + h1p[...]
    o1 = jnp.maximum(dinv[...] * s + b1[...], 0.0)
    h2p_o[...] = dinv[...] * jnp.dot(o1, w2[...],
                                     preferred_element_type=jnp.float32)


def _tc3_body(p2, h2p, dinv, b2, out_o):
    out_o[...] = dinv[...] * (p2[0] + p2[1] + h2p[...]) + b2[...]


_tc1 = pl.pallas_call(
    _tc1_body,
    out_shape=[
        jax.ShapeDtypeStruct((NPAD, 128), jnp.float32),
        jax.ShapeDtypeStruct((NPAD, 1), jnp.float32),
    ],
)

_tc2 = pl.pallas_call(
    _tc2_body,
    out_shape=jax.ShapeDtypeStruct((NPAD, 64), jnp.float32),
)

_tc3 = pl.pallas_call(
    _tc3_body,
    out_shape=jax.ShapeDtypeStruct((NPAD, 64), jnp.float32),
)


def kernel(x, edge_index, W1, b1, W2, b2):
    ei = edge_index.astype(jnp.int32)
    pad = EPAD - ei.shape[1]
    row = jnp.concatenate([ei[0], jnp.zeros((pad,), jnp.int32)])
    col = jnp.concatenate([ei[1], jnp.full((pad,), N, jnp.int32)])
    er = row.reshape(NW, CH, K)
    ec = col.reshape(NW, CH, K)

    x_pad = jnp.zeros((NPAD, 128), jnp.float32).at[:N].set(x)
    ones_kw = jnp.ones((K, DEGW), jnp.float32)
    zz16 = jnp.zeros((NPAD, DEGW), jnp.float32)
    zz128 = jnp.zeros((NPAD, 128), jnp.float32)
    zz64 = jnp.zeros((NPAD, 64), jnp.float32)
    b1r = b1.reshape(1, -1)
    b2r = b2.reshape(1, -1)

    degp = _deg_kernel(ec, ones_kw, zz16)
    h1p, dinv = _tc1(degp, x_pad, W1)
    p1 = _agg128(h1p, er, ec, zz128)
    h2p = _tc2(p1, h1p, dinv, b1r, W2)
    p2 = _agg64(h2p, er, ec, zz64)
    out = _tc3(p2, h2p, dinv, b2r)
    return out[:N]
